# Initial kernel scaffold; baseline (speedup 1.0000x reference)
import functools

import jax
import jax.numpy as jnp
from jax import lax
from jax.experimental import pallas as pl
from jax.experimental.pallas import tpu as pltpu
from jax.experimental.pallas import tpu_sc as plsc

N_ST = 100000
NNZ = 6400000
NC, NS, L = 2, 16, 16
NW = NC * NS
PAD = 100352
BLK = 2048
ROWS = BLK // 128
NBLK = NNZ // BLK
SLICE = PAD // NS

_mesh = plsc.VectorSubcoreMesh(
    core_axis_name="c", subcore_axis_name="s", num_cores=NC, num_subcores=NS
)


@functools.partial(
    pl.kernel,
    out_type=jax.ShapeDtypeStruct((NC, PAD), jnp.float32),
    mesh=_mesh,
    scratch_types=[
        pltpu.VMEM((N_ST,), jnp.float32),
        pltpu.VMEM((BLK,), jnp.int32),
        pltpu.VMEM((BLK,), jnp.float32),
        pltpu.VMEM((ROWS, 128), jnp.int32),
        pltpu.VMEM((ROWS, 128), jnp.float32),
        pltpu.VMEM((SLICE,), jnp.float32),
        pltpu.VMEM_SHARED((PAD,), jnp.float32),
        pltpu.SemaphoreType.DMA,
        pltpu.SemaphoreType.DMA,
        pltpu.SemaphoreType.DMA,
        pltpu.SemaphoreType.DMA,
    ],
)
def _seg_sum_sc(alpha_hbm, vals_hbm, from_hbm, to2d_hbm, out_hbm,
                alpha_v, f_v, v_v, t_v, e_v, z_v, acc_sh,
                sem_a, sem_b, sem_c, sem_d):
    c = lax.axis_index("c")
    s = lax.axis_index("s")
    w = c * NS + s

    cp_alpha = pltpu.async_copy(alpha_hbm, alpha_v, sem_a)

    zero = jnp.zeros((L,), jnp.float32)

    def _zero_body(i, carry):
        z_v[pl.ds(i * L, L)] = zero
        return carry

    lax.fori_loop(0, SLICE // L, _zero_body, 0)
    pltpu.sync_copy(z_v, acc_sh.at[pl.ds(s * SLICE, SLICE)])
    cp_alpha.wait()
    plsc.subcore_barrier()

    b0 = (w * NBLK) // NW
    b1 = ((w + 1) * NBLK) // NW

    def _block_body(blk, carry):
        base = blk * BLK
        cp_f = pltpu.async_copy(from_hbm.at[pl.ds(base, BLK)], f_v, sem_a)
        cp_v = pltpu.async_copy(vals_hbm.at[pl.ds(base, BLK)], v_v, sem_b)
        cp_t = pltpu.async_copy(to2d_hbm.at[pl.ds(blk * ROWS, ROWS)], t_v, sem_c)
        cp_f.wait()
        cp_v.wait()
        cp_t.wait()
        for r in range(ROWS):
            for cc in range(128 // L):
                i = r * (128 // L) + cc
                idxf = f_v[pl.ds(i * L, L)]
                a = plsc.load_gather(alpha_v, [idxf])
                u = a + v_v[pl.ds(i * L, L)]
                e_v[r, pl.ds(cc * L, L)] = jnp.exp(u)
        cps = [
            pltpu.async_copy(e_v.at[r], acc_sh.at[t_v.at[r]], sem_d, add=True)
            for r in range(ROWS)
        ]
        for cp in cps:
            cp.wait()
        return carry

    lax.fori_loop(b0, b1, _block_body, 0)

    plsc.subcore_barrier()
    pltpu.sync_copy(
        acc_sh.at[pl.ds(s * SLICE, SLICE)],
        out_hbm.at[c, pl.ds(s * SLICE, SLICE)],
    )


def _log_tc(p_ref, o_ref):
    seg = p_ref[0] + p_ref[1]
    o_ref[...] = jnp.log(jnp.maximum(seg, 1e-38))


_log_call = pl.pallas_call(
    _log_tc,
    out_shape=jax.ShapeDtypeStruct((PAD // 128, 128), jnp.float32),
)


@jax.jit
def kernel(log_alpha, log_vals, idx_from, idx_to):
    idx_from = idx_from.astype(jnp.int32)
    to2d = idx_to.astype(jnp.int32).reshape(NNZ // 128, 128)
    partials = _seg_sum_sc(log_alpha, log_vals, idx_from, to2d)
    out = _log_call(partials.reshape(NC, PAD // 128, 128))
    return out.reshape(-1)[:N_ST]


# speedup vs baseline: 320.5917x; 320.5917x over previous
import functools

import jax
import jax.numpy as jnp
from jax import lax
from jax.experimental import pallas as pl
from jax.experimental.pallas import tpu as pltpu
from jax.experimental.pallas import tpu_sc as plsc

N_ST = 100000
NNZ = 6400000
NC, NS, L = 2, 16, 16
NW = NC * NS
PAD = 100352
BLK = 2048
ROWS = BLK // 128
NBLK = NNZ // BLK
SLICE = PAD // NS

_mesh = plsc.VectorSubcoreMesh(
    core_axis_name="c", subcore_axis_name="s", num_cores=NC, num_subcores=NS
)


@functools.partial(
    pl.kernel,
    out_type=jax.ShapeDtypeStruct((NC, PAD), jnp.float32),
    mesh=_mesh,
    compiler_params=pltpu.CompilerParams(needs_layout_passes=False),
    scratch_types=[
        pltpu.VMEM((N_ST,), jnp.float32),
        pltpu.VMEM((BLK,), jnp.int32),
        pltpu.VMEM((BLK,), jnp.float32),
        pltpu.VMEM((ROWS, 128), jnp.int32),
        pltpu.VMEM((ROWS, 128), jnp.float32),
        pltpu.VMEM((SLICE,), jnp.float32),
        pltpu.VMEM_SHARED((PAD,), jnp.float32),
        pltpu.SemaphoreType.DMA,
        pltpu.SemaphoreType.DMA,
        pltpu.SemaphoreType.DMA,
        pltpu.SemaphoreType.DMA,
    ],
)
def _seg_sum_sc(alpha_hbm, vals_hbm, from_hbm, to2d_hbm, out_hbm,
                alpha_v, f_v, v_v, t_v, e_v, z_v, acc_sh,
                sem_a, sem_b, sem_c, sem_d):
    c = lax.axis_index("c")
    s = lax.axis_index("s")
    w = c * NS + s

    cp_alpha = pltpu.async_copy(alpha_hbm, alpha_v, sem_a)

    zero = jnp.zeros((L,), jnp.float32)

    def _zero_body(i, carry):
        z_v[pl.ds(i * L, L)] = zero
        return carry

    lax.fori_loop(0, SLICE // L, _zero_body, 0)
    pltpu.sync_copy(z_v, acc_sh.at[pl.ds(s * SLICE, SLICE)])
    cp_alpha.wait()
    plsc.subcore_barrier()

    b0 = (w * NBLK) // NW
    b1 = ((w + 1) * NBLK) // NW

    def _block_body(blk, carry):
        base = blk * BLK
        cp_f = pltpu.async_copy(from_hbm.at[pl.ds(base, BLK)], f_v, sem_a)
        cp_v = pltpu.async_copy(vals_hbm.at[pl.ds(base, BLK)], v_v, sem_b)
        cp_t = pltpu.async_copy(to2d_hbm.at[pl.ds(blk * ROWS, ROWS)], t_v, sem_c)
        cp_f.wait()
        cp_v.wait()
        cp_t.wait()
        for r in range(ROWS):
            for cc in range(128 // L):
                i = r * (128 // L) + cc
                idxf = f_v[pl.ds(i * L, L)]
                a = plsc.load_gather(alpha_v, [idxf])
                u = a + v_v[pl.ds(i * L, L)]
                e_v[r, pl.ds(cc * L, L)] = jnp.exp(u)
        cps = [
            pltpu.async_copy(e_v.at[r], acc_sh.at[t_v.at[r]], sem_d, add=True)
            for r in range(ROWS)
        ]
        for cp in cps:
            cp.wait()
        return carry

    lax.fori_loop(b0, b1, _block_body, 0)

    plsc.subcore_barrier()
    pltpu.sync_copy(
        acc_sh.at[pl.ds(s * SLICE, SLICE)],
        out_hbm.at[c, pl.ds(s * SLICE, SLICE)],
    )


def _log_tc(p_ref, o_ref):
    seg = p_ref[0] + p_ref[1]
    o_ref[...] = jnp.log(jnp.maximum(seg, 1e-38))


_log_call = pl.pallas_call(
    _log_tc,
    out_shape=jax.ShapeDtypeStruct((PAD // 128, 128), jnp.float32),
)


@jax.jit
def kernel(log_alpha, log_vals, idx_from, idx_to):
    idx_from = idx_from.astype(jnp.int32)
    to2d = idx_to.astype(jnp.int32).reshape(NNZ // 128, 128)
    partials = _seg_sum_sc(log_alpha, log_vals, idx_from, to2d)
    out = _log_call(partials.reshape(NC, PAD // 128, 128))
    return out.reshape(-1)[:N_ST]
